# Initial kernel scaffold; baseline (speedup 1.0000x reference)
import jax
import jax.numpy as jnp
from jax.experimental import pallas as pl


def kernel(indices, prob, label, text_prompt, nc_token_prefix, nc_token_suffix):
    raise NotImplementedError("write your pallas kernel here")


# speedup vs baseline: 1.4737x; 1.4737x over previous
import functools

import jax
import jax.numpy as jnp
from jax import lax
from jax.experimental import pallas as pl
from jax.experimental.pallas import tpu as pltpu
from jax.experimental.pallas import tpu_sc as plsc

_N_CLS = 100
_NUM_PROMPT = 16
_N_CTX = 16
_D = 768
_B = 1024
_K = 4
_ROW = _N_CTX * _D
_TOK = 2 + _K * _N_CTX
_L = 16

_NC = 2
_NS = 16
_NW = _NC * _NS
_BPW = _B // _NW
_PAIRS = _BPW * _K
_GRP = 8
_NGRP = _PAIRS // _GRP


def _sc_body(tab, pre, suf, idxf, probf, lab, out,
             label_v, ind_v, prob_v, gat_v, buf, ps_buf, sem):
    wid = lax.axis_index("s") * _NC + lax.axis_index("c")
    base = wid * _BPW

    pltpu.sync_copy(lab.at[pl.ds(base, _BPW)], label_v)
    pltpu.sync_copy(idxf.at[pl.ds(base * _K, _PAIRS)], ind_v)
    pltpu.sync_copy(probf.at[pl.ds(base * _K, _PAIRS)], prob_v)

    for i in range(_PAIRS // _L):
        lanes = lax.iota(jnp.int32, _L) + i * _L
        b_ix = lax.shift_right_logical(lanes, 2)
        labv = plsc.load_gather(label_v, [b_ix])
        indv = ind_v[pl.ds(i * _L, _L)]
        gat_v[pl.ds(i * _L, _L)] = labv * _NUM_PROMPT + indv

    pltpu.async_copy(pre.at[label_v], ps_buf, sem).wait()

    def pre_body(j, c):
        pltpu.sync_copy(ps_buf.at[j], out.at[pl.ds((base + j) * _TOK * _D, _D)])
        return c

    lax.fori_loop(0, _BPW, pre_body, 0)

    pltpu.async_copy(suf.at[label_v], ps_buf, sem).wait()

    def suf_body(j, c):
        pltpu.sync_copy(
            ps_buf.at[j],
            out.at[pl.ds(((base + j) * _TOK + _TOK - 1) * _D, _D)])
        return c

    lax.fori_loop(0, _BPW, suf_body, 0)

    def grp_body(g, c):
        pltpu.async_copy(tab.at[gat_v.at[pl.ds(g * _GRP, _GRP)]], buf, sem).wait()
        for q in range(_GRP):
            p = g * _GRP + q
            pv = plsc.load_gather(prob_v, [lax.broadcast(p, (_L,))])

            def sc_body(j, cc):
                buf[q, pl.ds(j * _L, _L)] = buf[q, pl.ds(j * _L, _L)] * pv
                return cc

            lax.fori_loop(0, _ROW // _L, sc_body, 0)
            b = base + lax.shift_right_logical(p, 2)
            k = lax.bitwise_and(p, _K - 1)
            off = (b * _TOK + 1 + k * _N_CTX) * _D
            pltpu.sync_copy(buf.at[q], out.at[pl.ds(off, _ROW)])
        return c

    lax.fori_loop(0, _NGRP, grp_body, 0)


@jax.jit
def _prompt_gather(table, pre, suf, ind_flat, prob_flat, label):
    mesh = plsc.VectorSubcoreMesh(core_axis_name="c", subcore_axis_name="s")
    fn = pl.kernel(
        _sc_body,
        out_type=jax.ShapeDtypeStruct((_B * _TOK * _D,), jnp.float32),
        mesh=mesh,
        compiler_params=pltpu.CompilerParams(needs_layout_passes=False),
        scratch_types=[
            pltpu.VMEM((_BPW,), jnp.int32),
            pltpu.VMEM((_PAIRS,), jnp.int32),
            pltpu.VMEM((_PAIRS,), jnp.float32),
            pltpu.VMEM((_PAIRS,), jnp.int32),
            pltpu.VMEM((_GRP, _ROW), jnp.float32),
            pltpu.VMEM((_BPW, _D), jnp.float32),
            pltpu.SemaphoreType.DMA,
        ],
    )
    return fn(table, pre, suf, ind_flat, prob_flat, label)


def kernel(indices, prob, label, text_prompt, nc_token_prefix, nc_token_suffix):
    table = text_prompt.reshape(_N_CLS * _NUM_PROMPT, _ROW)
    pre = nc_token_prefix.reshape(_N_CLS, _D)
    suf = nc_token_suffix.reshape(_N_CLS, _D)
    out_flat = _prompt_gather(table, pre, suf,
                              indices.reshape(-1).astype(jnp.int32),
                              prob.reshape(-1),
                              label.astype(jnp.int32))
    return out_flat.reshape(_B, _TOK, _D)
